# Initial kernel scaffold; baseline (speedup 1.0000x reference)
import jax
import jax.numpy as jnp
from jax.experimental import pallas as pl


def kernel(x, W1, b1, g1, be1, W2, b2, g2, be2, codebooks, W3, b3, g3, be3, W4, b4, g4, be4):
    raise NotImplementedError("write your pallas kernel here")


# speedup vs baseline: 1.2541x; 1.2541x over previous
import jax
import jax.numpy as jnp
from jax.experimental import pallas as pl
from jax.experimental.pallas import tpu as pltpu

_FIRST = 2048
_SECOND = 1024
_NQ = 8
_CB = 1024
_EMB = 4096
_TOK = 256


def _ln_act(h, g, be):
    mu = jnp.mean(h, axis=-1, keepdims=True)
    var = jnp.var(h, axis=-1, keepdims=True)
    h = (h - mu) / jnp.sqrt(var + 1e-5) * g + be
    return jnp.where(h >= 0, h, 0.01 * h)


def _mlp_kernel(x_ref, wa_ref, ba_ref, ga_ref, bea_ref, wb_ref, bb_ref,
                gb_ref, beb_ref, o_ref):
    h = jnp.dot(x_ref[...], wa_ref[...], preferred_element_type=jnp.float32)
    h = _ln_act(h + ba_ref[...], ga_ref[...], bea_ref[...])
    h = jnp.dot(h, wb_ref[...], preferred_element_type=jnp.float32)
    o_ref[...] = _ln_act(h + bb_ref[...], gb_ref[...], beb_ref[...])


def _vq_kernel(ze_ref, cb_ref, zq_ref, idx_ref, loss_ref, cbsq_ref):
    step = pl.program_id(0)
    nsteps = pl.num_programs(0)

    @pl.when(step == 0)
    def _init():
        cbsq_ref[...] = jnp.sum(cb_ref[...] ** 2, axis=-1)
        for q in range(_NQ):
            loss_ref[0, q] = 0.0

    r = ze_ref[...]
    qsum = jnp.zeros_like(r)
    for q in range(_NQ):
        cb = cb_ref[q]
        rsq = jnp.sum(r * r, axis=-1, keepdims=True)
        s = jax.lax.dot_general(r, cb, (((1,), (1,)), ((), ())),
                                preferred_element_type=jnp.float32)
        d = rsq - 2.0 * s + cbsq_ref[q][None, :]
        idx = jnp.argmin(d, axis=-1)
        dmin = jnp.min(d, axis=-1)
        oh = (jax.lax.broadcasted_iota(jnp.int32, d.shape, 1)
              == idx[:, None]).astype(jnp.float32)
        quant = jnp.dot(oh, cb, preferred_element_type=jnp.float32,
                        precision=jax.lax.Precision.HIGHEST)
        qsum = qsum + quant
        r = r - quant
        idx_ref[q, :] = idx
        loss_ref[0, q] = loss_ref[0, q] + jnp.sum(dmin)

    zq_ref[...] = qsum

    @pl.when(step == nsteps - 1)
    def _final():
        scale = 1.0 / (nsteps * _TOK * _SECOND)
        for q in range(_NQ):
            loss_ref[0, q] = loss_ref[0, q] * scale


def _row(v):
    return v.reshape(1, -1)


def _mlp_call(x, wa, ba, ga, bea, wb, bb, gb, beb):
    n, din = x.shape
    dmid = wa.shape[1]
    dout = wb.shape[1]
    grid = (n // _TOK,)
    return pl.pallas_call(
        _mlp_kernel,
        grid=grid,
        in_specs=[
            pl.BlockSpec((_TOK, din), lambda i: (i, 0)),
            pl.BlockSpec((din, dmid), lambda i: (0, 0)),
            pl.BlockSpec((1, dmid), lambda i: (0, 0)),
            pl.BlockSpec((1, dmid), lambda i: (0, 0)),
            pl.BlockSpec((1, dmid), lambda i: (0, 0)),
            pl.BlockSpec((dmid, dout), lambda i: (0, 0)),
            pl.BlockSpec((1, dout), lambda i: (0, 0)),
            pl.BlockSpec((1, dout), lambda i: (0, 0)),
            pl.BlockSpec((1, dout), lambda i: (0, 0)),
        ],
        out_specs=pl.BlockSpec((_TOK, dout), lambda i: (i, 0)),
        out_shape=jax.ShapeDtypeStruct((n, dout), jnp.float32),
        compiler_params=pltpu.CompilerParams(
            dimension_semantics=("arbitrary",),
            vmem_limit_bytes=63 * 1024 * 1024,
        ),
    )(x, wa, _row(ba), _row(ga), _row(bea), wb, _row(bb), _row(gb), _row(beb))


def _vq_call(z_e, codebooks):
    n = z_e.shape[0]
    grid = (n // _TOK,)
    z_q, idx, loss = pl.pallas_call(
        _vq_kernel,
        grid=grid,
        in_specs=[
            pl.BlockSpec((_TOK, _SECOND), lambda i: (i, 0)),
            pl.BlockSpec((_NQ, _CB, _SECOND), lambda i: (0, 0, 0)),
        ],
        out_specs=[
            pl.BlockSpec((_TOK, _SECOND), lambda i: (i, 0)),
            pl.BlockSpec((_NQ, _TOK), lambda i: (0, i)),
            pl.BlockSpec((1, _NQ), lambda i: (0, 0), memory_space=pltpu.SMEM),
        ],
        out_shape=[
            jax.ShapeDtypeStruct((n, _SECOND), jnp.float32),
            jax.ShapeDtypeStruct((_NQ, n), jnp.int32),
            jax.ShapeDtypeStruct((1, _NQ), jnp.float32),
        ],
        scratch_shapes=[pltpu.VMEM((_NQ, _CB), jnp.float32)],
        compiler_params=pltpu.CompilerParams(
            dimension_semantics=("arbitrary",),
            vmem_limit_bytes=63 * 1024 * 1024,
        ),
    )(z_e, codebooks)
    return z_q, idx, loss


def kernel(x, W1, b1, g1, be1, W2, b2, g2, be2, codebooks,
           W3, b3, g3, be3, W4, b4, g4, be4):
    batch, seq, emb = x.shape
    n = batch * seq
    xf = x.reshape(n, emb)

    z_e = _mlp_call(xf, W1, b1, g1, be1, W2, b2, g2, be2)
    z_q, idx, loss = _vq_call(z_e, codebooks)
    out = _mlp_call(z_q, W3, b3, g3, be3, W4, b4, g4, be4)

    out = out.reshape(batch, seq, emb)
    indices = idx.T.reshape(batch, seq, _NQ)
    cmt_loss = loss.reshape(_NQ)
    return (out, indices, cmt_loss)
